# baseline (speedup 1.0000x reference)
import jax
import jax.numpy as jnp
from jax import lax
from jax.experimental import pallas as pl
from jax.experimental.pallas import tpu as pltpu
from jax.experimental.pallas import tpu_sc as plsc

B = 16384
NF = 26
NN = 13
NR = NF + NN
K = 64

NC, NS = 2, 16
NW = NC * NS
BPW = B // NW
CB = 16
NCHUNK = BPW // CB
ROWS = CB * NR
IDXW = 104
NGAT = ROWS // IDXW


IDX_PW = BPW * NR // IDXW
NF_PW = BPW * NN


def _sc_body(idx_hbm, nf_hbm, table_hbm, net_hbm, out_hbm,
             idx_v, nf_v, net_v, stage, gsem):
    wid = lax.axis_index("s") * NC + lax.axis_index("c")
    pltpu.sync_copy(net_hbm, net_v)
    pltpu.sync_copy(idx_hbm.at[pl.ds(wid * IDX_PW, IDX_PW), :], idx_v)
    pltpu.sync_copy(nf_hbm.at[pl.ds(wid * NF_PW, NF_PW)],
                    nf_v.at[pl.ds(0, NF_PW)])

    def chunk(c, carry):
        r0 = (wid * BPW + c * CB) * NR
        copies = [
            pltpu.async_copy(
                table_hbm.at[plsc.Indices(idx_v.at[c * NGAT + g],
                                          ignored_value=-1)],
                stage.at[pl.ds(g * IDXW, IDXW), :],
                gsem,
            )
            for g in range(NGAT)
        ]

        def nbody(i, c2):
            vals = nf_v[pl.ds((c * CB + i) * NN, 16)]
            for k in range(NN):
                src = vals[k]
                row = i * NR + NF + k
                for jc in range(K // 16):
                    stage[row, pl.ds(jc * 16, 16)] = (
                        src * net_v[pl.ds(k * K + jc * 16, 16)])
            return c2

        lax.fori_loop(0, CB, nbody, 0)
        for cp in copies:
            cp.wait()
        pltpu.sync_copy(stage, out_hbm.at[pl.ds(r0, ROWS), :])
        return carry

    lax.fori_loop(0, NCHUNK, chunk, 0)


def kernel(cat_features, num_features, cat_table, num_embedding):
    cat = cat_features.astype(jnp.int32)
    idx = jnp.concatenate([cat, jnp.full((B, NN), -1, jnp.int32)], axis=1)
    idx = idx.reshape(B * NR // IDXW, IDXW)
    nf = num_features.reshape(B * NN)
    net = num_embedding.T.reshape(NN * K)
    mesh = plsc.VectorSubcoreMesh(core_axis_name="c", subcore_axis_name="s")
    f = pl.kernel(
        _sc_body,
        out_type=jax.ShapeDtypeStruct((B * NR, K), jnp.float32),
        mesh=mesh,
        compiler_params=pltpu.CompilerParams(use_tc_tiling_on_sc=False),
        scratch_types=[
            pltpu.VMEM((IDX_PW, IDXW), jnp.int32),
            pltpu.VMEM((NF_PW + 16,), jnp.float32),
            pltpu.VMEM((NN * K,), jnp.float32),
            pltpu.VMEM((ROWS, K), jnp.float32),
            pltpu.SemaphoreType.DMA,
        ],
    )
    out = f(idx, nf, cat_table, net)
    return out.reshape(B, NR, K)


# speedup vs baseline: 1.0175x; 1.0175x over previous
import jax
import jax.numpy as jnp
from jax import lax
from jax.experimental import pallas as pl
from jax.experimental.pallas import tpu as pltpu
from jax.experimental.pallas import tpu_sc as plsc

B = 16384
NF = 26
NN = 13
NR = NF + NN
K = 64

NC, NS = 2, 16
NW = NC * NS
BPW = B // NW
NBB = BPW // 128
FS = 2
NCATC = NBB * (NF // FS)
NNUMC = NBB * NN
IDX_ROWS = NF * NBB
NF_PW = NN * BPW
ROW_RUN = 8 * 128
FBLK = FS * K * 128
OUT_STRIDE = 128 * ROW_RUN


def _transpose_block(rows_v, stage, frel):
    base0 = frel * FS * 4096

    def body(b0, carry):
        for kc in range(K // 16):
            idx = carry + jnp.full((16,), kc * 16 * 128, jnp.int32)
            v = rows_v[frel * 128 + b0, pl.ds(kc * 16, 16)]
            plsc.store_scatter(stage, [idx], v)
        return carry + jnp.full((16,), 1, jnp.int32)

    init = lax.iota(jnp.int32, 16) * 128 + base0
    lax.fori_loop(0, 128, body, init)


def _sc_body(idx_hbm, nf_hbm, table_hbm, net_hbm, out_hbm,
             idx_v, nf_v, net_v, rows0, rows1, stage0, stage1,
             gs0, gs1, os0, os1):
    wid = lax.axis_index("s") * NC + lax.axis_index("c")
    pltpu.sync_copy(net_hbm, net_v)
    pltpu.sync_copy(idx_hbm.at[pl.ds(wid * IDX_ROWS, IDX_ROWS), :], idx_v)
    pltpu.sync_copy(nf_hbm.at[pl.ds(wid * NF_PW, NF_PW)], nf_v)
    rows = (rows0, rows1)
    stages = (stage0, stage1)
    gsems = (gs0, gs1)
    osems = (os0, os1)

    def cat_gathers(t, slot):
        bbl = t // (NF // FS)
        fc = t % (NF // FS)
        return [
            pltpu.make_async_copy(
                table_hbm.at[idx_v.at[(fc * FS + frel) * NBB + bbl]],
                rows[slot].at[pl.ds(frel * 128, 128), :],
                gsems[slot],
            )
            for frel in range(FS)
        ]

    def out_copies(f0, bbg, stage, osem, nf):
        return [
            pltpu.make_async_copy(
                stage.at[pl.ds(frel * K * 128 + k8 * ROW_RUN, ROW_RUN)],
                out_hbm.at[pl.ds(
                    ((f0 + frel) * 8 + k8) * OUT_STRIDE + bbg * ROW_RUN,
                    ROW_RUN)],
                osem,
            )
            for frel in range(nf)
            for k8 in range(8)
        ]

    for cp in cat_gathers(0, 0):
        cp.start()
    for cp in cat_gathers(1, 1):
        cp.start()

    def cat_chunk(i, slot):
        t = 2 * i + slot
        bbl = t // (NF // FS)
        fc = t % (NF // FS)
        f0 = fc * FS
        bbg = wid * NBB + bbl
        for cp in cat_gathers(t, slot):
            cp.wait()

        @pl.when(i >= 1)
        def _():
            pltpu.make_async_copy(
                out_hbm.at[pl.ds(0, FBLK)], stages[slot], osems[slot]
            ).wait()

        for frel in range(FS):
            _transpose_block(rows[slot], stages[slot], frel)
        for cp in out_copies(f0, bbg, stages[slot], osems[slot], FS):
            cp.start()

        @pl.when(t + 2 < NCATC)
        def _():
            for cp in cat_gathers(t + 2, slot):
                cp.start()

    def cat_loop(i, carry):
        cat_chunk(i, 0)
        cat_chunk(i, 1)
        return carry

    lax.fori_loop(0, NCATC // 2, cat_loop, 0)
    for slot in range(2):
        pltpu.make_async_copy(
            out_hbm.at[pl.ds(0, FBLK)], stages[slot], osems[slot]).wait()

    def num_chunk(i, slot):
        t = 2 * i + slot
        bbl = t // NN
        j = t % NN
        bbg = wid * NBB + bbl
        stage = stages[slot]

        @pl.when(i >= 1)
        def _():
            pltpu.make_async_copy(
                out_hbm.at[pl.ds(0, K * 128)],
                stage.at[pl.ds(0, K * 128)], osems[slot]).wait()

        def nbody(b16, carry):
            nfvec = nf_v[pl.ds(j * BPW + bbl * 128 + b16 * 16, 16)]
            for kc in range(K // 16):
                netw = net_v[pl.ds(j * K + kc * 16, 16)]
                for k0 in range(16):
                    k = kc * 16 + k0
                    stage[pl.ds(k * 128 + b16 * 16, 16)] = netw[k0] * nfvec
            return carry

        lax.fori_loop(0, 8, nbody, 0)
        for cp in out_copies(NF + j, bbg, stage, osems[slot], 1):
            cp.start()

    def num_loop(i, carry):
        num_chunk(i, 0)
        num_chunk(i, 1)
        return carry

    lax.fori_loop(0, NNUMC // 2, num_loop, 0)
    for slot in range(2):
        pltpu.make_async_copy(
            out_hbm.at[pl.ds(0, K * 128)],
            stages[slot].at[pl.ds(0, K * 128)], osems[slot]).wait()


def kernel(cat_features, num_features, cat_table, num_embedding):
    cat = cat_features.astype(jnp.int32)
    idx = (cat.T.reshape(NF, NW, NBB, 128)
           .transpose(1, 0, 2, 3).reshape(NW * IDX_ROWS, 128))
    nf = (num_features.T.reshape(NN, NW, NBB * 128)
          .transpose(1, 0, 2).reshape(NW * NF_PW))
    net = num_embedding.T.reshape(NN * K)
    mesh = plsc.VectorSubcoreMesh(core_axis_name="c", subcore_axis_name="s")
    f = pl.kernel(
        _sc_body,
        out_type=jax.ShapeDtypeStruct((NR * K * B,), jnp.float32),
        mesh=mesh,
        compiler_params=pltpu.CompilerParams(
            use_tc_tiling_on_sc=False, needs_layout_passes=False),
        scratch_types=[
            pltpu.VMEM((IDX_ROWS, 128), jnp.int32),
            pltpu.VMEM((NF_PW,), jnp.float32),
            pltpu.VMEM((NN * K,), jnp.float32),
            pltpu.VMEM((FS * 128, K), jnp.float32),
            pltpu.VMEM((FS * 128, K), jnp.float32),
            pltpu.VMEM((FBLK,), jnp.float32),
            pltpu.VMEM((FBLK,), jnp.float32),
            pltpu.SemaphoreType.DMA,
            pltpu.SemaphoreType.DMA,
            pltpu.SemaphoreType.DMA,
            pltpu.SemaphoreType.DMA,
        ],
    )
    out_flat = f(idx, nf, cat_table, net)
    out5 = out_flat.reshape(NR, 8, B // 128, 8, 128)
    return out5.transpose(2, 4, 0, 1, 3).reshape(B, NR, K)


# speedup vs baseline: 1.0187x; 1.0012x over previous
import functools

import jax
import jax.numpy as jnp
from jax import lax
from jax.experimental import pallas as pl
from jax.experimental.pallas import tpu as pltpu
from jax.experimental.pallas import tpu_sc as plsc

B = 16384
NF = 26
NN = 13
NR = NF + NN
K = 64

NC, NS = 2, 16
NW = NC * NS
BPW = B // NW
NBB = BPW // 128
FS = 2
NCATC = NBB * (NF // FS)
NNUMC = NBB * NN
IDX_ROWS = NF * NBB
NF_PW = NN * BPW
ROW_RUN = 8 * 128
FBLK = FS * K * 128
OUT_STRIDE = 128 * ROW_RUN


def _transpose_block(rows_v, stage, frel):
    base0 = frel * K * 128
    iota128 = lax.iota(jnp.int32, 16) * 128

    def body(i, carry):
        b0 = i * 4
        for u in range(4):
            base = iota128 + (base0 + b0 + u)
            for kc in range(K // 16):
                v = rows_v[frel * 128 + b0 + u, pl.ds(kc * 16, 16)]
                plsc.store_scatter(stage, [base + kc * 16 * 128], v)
        return carry

    lax.fori_loop(0, 32, body, 0)


def _sc_body(idx_hbm, nf_hbm, table_hbm, net_hbm, out_hbm,
             idx_v, nf_v, net_v, rows0, rows1, stage0, stage1,
             gs0, gs1, os0, os1):
    wid = lax.axis_index("s") * NC + lax.axis_index("c")
    pltpu.sync_copy(net_hbm, net_v)
    pltpu.sync_copy(idx_hbm.at[pl.ds(wid * IDX_ROWS, IDX_ROWS), :], idx_v)
    pltpu.sync_copy(nf_hbm.at[pl.ds(wid * NF_PW, NF_PW)], nf_v)
    rows = (rows0, rows1)
    stages = (stage0, stage1)
    gsems = (gs0, gs1)
    osems = (os0, os1)

    def cat_gathers(t, slot):
        bbl = t // (NF // FS)
        fc = t % (NF // FS)
        return [
            pltpu.make_async_copy(
                table_hbm.at[idx_v.at[(fc * FS + frel) * NBB + bbl]],
                rows[slot].at[pl.ds(frel * 128, 128), :],
                gsems[slot],
            )
            for frel in range(FS)
        ]

    def out_copies(f0, bbg, stage, osem, nf):
        return [
            pltpu.make_async_copy(
                stage.at[pl.ds(frel * K * 128 + k8 * ROW_RUN, ROW_RUN)],
                out_hbm.at[pl.ds(
                    ((f0 + frel) * 8 + k8) * OUT_STRIDE + bbg * ROW_RUN,
                    ROW_RUN)],
                osem,
            )
            for frel in range(nf)
            for k8 in range(8)
        ]

    for cp in cat_gathers(0, 0):
        cp.start()
    for cp in cat_gathers(1, 1):
        cp.start()

    def cat_chunk(i, slot):
        t = 2 * i + slot
        bbl = t // (NF // FS)
        fc = t % (NF // FS)
        f0 = fc * FS
        bbg = wid * NBB + bbl
        for cp in cat_gathers(t, slot):
            cp.wait()

        @pl.when(i >= 1)
        def _():
            pltpu.make_async_copy(
                out_hbm.at[pl.ds(0, FBLK)], stages[slot], osems[slot]
            ).wait()

        for frel in range(FS):
            _transpose_block(rows[slot], stages[slot], frel)
        for cp in out_copies(f0, bbg, stages[slot], osems[slot], FS):
            cp.start()

        @pl.when(t + 2 < NCATC)
        def _():
            for cp in cat_gathers(t + 2, slot):
                cp.start()

    def cat_loop(i, carry):
        cat_chunk(i, 0)
        cat_chunk(i, 1)
        return carry

    lax.fori_loop(0, NCATC // 2, cat_loop, 0)
    for slot in range(2):
        pltpu.make_async_copy(
            out_hbm.at[pl.ds(0, FBLK)], stages[slot], osems[slot]).wait()

    def num_chunk(i, slot):
        t = 2 * i + slot
        bbl = t // NN
        j = t % NN
        bbg = wid * NBB + bbl
        stage = stages[slot]

        @pl.when(i >= 1)
        def _():
            pltpu.make_async_copy(
                out_hbm.at[pl.ds(0, K * 128)],
                stage.at[pl.ds(0, K * 128)], osems[slot]).wait()

        netws = [net_v[pl.ds(j * K + kc * 16, 16)] for kc in range(K // 16)]

        def nbody(b16, carry):
            nfvec = nf_v[pl.ds(j * BPW + bbl * 128 + b16 * 16, 16)]
            for kc in range(K // 16):
                for k0 in range(16):
                    k = kc * 16 + k0
                    stage[pl.ds(k * 128 + b16 * 16, 16)] = (
                        netws[kc][k0] * nfvec)
            return carry

        lax.fori_loop(0, 8, nbody, 0)
        for cp in out_copies(NF + j, bbg, stage, osems[slot], 1):
            cp.start()

    def num_loop(i, carry):
        num_chunk(i, 0)
        num_chunk(i, 1)
        return carry

    lax.fori_loop(0, NNUMC // 2, num_loop, 0)
    for slot in range(2):
        pltpu.make_async_copy(
            out_hbm.at[pl.ds(0, K * 128)],
            stages[slot].at[pl.ds(0, K * 128)], osems[slot]).wait()


def kernel(cat_features, num_features, cat_table, num_embedding):
    cat = cat_features.astype(jnp.int32)
    idx = (cat.T.reshape(NF, NW, NBB, 128)
           .transpose(1, 0, 2, 3).reshape(NW * IDX_ROWS, 128))
    nf = (num_features.T.reshape(NN, NW, NBB * 128)
          .transpose(1, 0, 2).reshape(NW * NF_PW))
    net = num_embedding.T.reshape(NN * K)
    mesh = plsc.VectorSubcoreMesh(core_axis_name="c", subcore_axis_name="s")
    f = pl.kernel(
        _sc_body,
        out_type=jax.ShapeDtypeStruct((NR * K * B,), jnp.float32),
        mesh=mesh,
        compiler_params=pltpu.CompilerParams(
            use_tc_tiling_on_sc=False, needs_layout_passes=False),
        scratch_types=[
            pltpu.VMEM((IDX_ROWS, 128), jnp.int32),
            pltpu.VMEM((NF_PW,), jnp.float32),
            pltpu.VMEM((NN * K,), jnp.float32),
            pltpu.VMEM((FS * 128, K), jnp.float32),
            pltpu.VMEM((FS * 128, K), jnp.float32),
            pltpu.VMEM((FBLK,), jnp.float32),
            pltpu.VMEM((FBLK,), jnp.float32),
            pltpu.SemaphoreType.DMA,
            pltpu.SemaphoreType.DMA,
            pltpu.SemaphoreType.DMA,
            pltpu.SemaphoreType.DMA,
        ],
    )
    out_flat = f(idx, nf, cat_table, net)
    out5 = out_flat.reshape(NR, 8, B // 128, 8, 128)
    return out5.transpose(2, 4, 0, 1, 3).reshape(B, NR, K)


# speedup vs baseline: 1.0202x; 1.0015x over previous
import jax
import jax.numpy as jnp
from jax import lax
from jax.experimental import pallas as pl
from jax.experimental.pallas import tpu as pltpu
from jax.experimental.pallas import tpu_sc as plsc

B = 16384
NF = 26
NN = 13
NR = NF + NN
K = 64

NC, NS = 2, 16
NW = NC * NS
BPW = B // NW
NBB = BPW // 128
FS = 2
NCATC = NBB * (NF // FS)
NNUMC = NBB * NN
IDX_ROWS = NF * NBB
NF_PW = NN * BPW
ROW_RUN = 8 * 128
OUT_STRIDE = 128 * ROW_RUN


def _transpose_block(rows_v, stage4, frel):
    iota = lax.iota(jnp.int32, 16)
    i_frel = jnp.full((16,), frel, jnp.int32)
    i_k8 = [(kc * 16 + iota) // 8 for kc in range(K // 16)]
    i_k0 = iota % 8

    def body(i, carry):
        b0 = i * 4
        for u in range(4):
            i_b = jnp.full((16,), b0 + u, jnp.int32)
            for kc in range(K // 16):
                v = rows_v[frel * 128 + b0 + u, pl.ds(kc * 16, 16)]
                plsc.store_scatter(stage4, [i_frel, i_k8[kc], i_k0, i_b], v)
        return carry

    lax.fori_loop(0, 32, body, 0)


def _sc_body(idx_hbm, nf_hbm, table_hbm, net_hbm, out_hbm,
             idx_v, nf_v, net_v, rows0, rows1, stage0, stage1,
             gs0, gs1, os0, os1):
    wid = lax.axis_index("s") * NC + lax.axis_index("c")
    pltpu.sync_copy(net_hbm, net_v)
    pltpu.sync_copy(idx_hbm.at[pl.ds(wid * IDX_ROWS, IDX_ROWS), :], idx_v)
    pltpu.sync_copy(nf_hbm.at[pl.ds(wid * NF_PW, NF_PW)], nf_v)
    rows = (rows0, rows1)
    stages = (stage0, stage1)
    gsems = (gs0, gs1)
    osems = (os0, os1)

    def cat_gathers(t, slot):
        bbl = t // (NF // FS)
        fc = t % (NF // FS)
        return [
            pltpu.make_async_copy(
                table_hbm.at[idx_v.at[(fc * FS + frel) * NBB + bbl]],
                rows[slot].at[pl.ds(frel * 128, 128), :],
                gsems[slot],
            )
            for frel in range(FS)
        ]

    def cat_out(f0, bbg, stage4, osem):
        return pltpu.make_async_copy(
            stage4,
            out_hbm.at[pl.ds(f0, FS), :, bbg, :, :],
            osem,
        )

    for cp in cat_gathers(0, 0):
        cp.start()
    for cp in cat_gathers(1, 1):
        cp.start()

    def cat_chunk(i, slot):
        t = 2 * i + slot
        bbl = t // (NF // FS)
        fc = t % (NF // FS)
        f0 = fc * FS
        bbg = wid * NBB + bbl
        for cp in cat_gathers(t, slot):
            cp.wait()

        @pl.when(i >= 1)
        def _():
            pltpu.make_async_copy(
                out_hbm.at[pl.ds(0, FS), :, 0, :, :],
                stages[slot], osems[slot]).wait()

        for frel in range(FS):
            _transpose_block(rows[slot], stages[slot], frel)
        cat_out(f0, bbg, stages[slot], osems[slot]).start()

        @pl.when(t + 2 < NCATC)
        def _():
            for cp in cat_gathers(t + 2, slot):
                cp.start()

    def cat_loop(i, carry):
        cat_chunk(i, 0)
        cat_chunk(i, 1)
        return carry

    lax.fori_loop(0, NCATC // 2, cat_loop, 0)
    for slot in range(2):
        pltpu.make_async_copy(
            out_hbm.at[pl.ds(0, FS), :, 0, :, :],
            stages[slot], osems[slot]).wait()

    def num_chunk(i, slot):
        t = 2 * i + slot
        bbl = t // NN
        j = t % NN
        bbg = wid * NBB + bbl
        stage2 = stages[slot]

        @pl.when(i >= 1)
        def _():
            pltpu.make_async_copy(
                out_hbm.at[0, :, 0, :, :],
                stage2.at[0], osems[slot]).wait()

        netws = [net_v[pl.ds(j * K + kc * 16, 16)] for kc in range(K // 16)]

        def nbody(b16, carry):
            nfvec = nf_v[pl.ds(j * BPW + bbl * 128 + b16 * 16, 16)]
            for k in range(K):
                stage2[0, k // 8, k % 8, pl.ds(b16 * 16, 16)] = (
                    netws[k // 16][k % 16] * nfvec)
            return carry

        lax.fori_loop(0, 8, nbody, 0)
        pltpu.make_async_copy(
            stage2.at[0],
            out_hbm.at[NF + j, :, bbg, :, :],
            osems[slot]).start()

    def num_loop(i, carry):
        num_chunk(i, 0)
        num_chunk(i, 1)
        return carry

    lax.fori_loop(0, NNUMC // 2, num_loop, 0)
    for slot in range(2):
        pltpu.make_async_copy(
            out_hbm.at[0, :, 0, :, :],
            stages[slot].at[0], osems[slot]).wait()


def kernel(cat_features, num_features, cat_table, num_embedding):
    cat = cat_features.astype(jnp.int32)
    idx = (cat.T.reshape(NF, NW, NBB, 128)
           .transpose(1, 0, 2, 3).reshape(NW * IDX_ROWS, 128))
    nf = (num_features.T.reshape(NN, NW, NBB * 128)
          .transpose(1, 0, 2).reshape(NW * NF_PW))
    net = num_embedding.T.reshape(NN * K)
    mesh = plsc.VectorSubcoreMesh(core_axis_name="c", subcore_axis_name="s")
    f = pl.kernel(
        _sc_body,
        out_type=jax.ShapeDtypeStruct((NR, 8, B // 128, 8, 128),
                                      jnp.float32),
        mesh=mesh,
        compiler_params=pltpu.CompilerParams(
            use_tc_tiling_on_sc=False, needs_layout_passes=False),
        scratch_types=[
            pltpu.VMEM((IDX_ROWS, 128), jnp.int32),
            pltpu.VMEM((NF_PW,), jnp.float32),
            pltpu.VMEM((NN * K,), jnp.float32),
            pltpu.VMEM((FS * 128, K), jnp.float32),
            pltpu.VMEM((FS * 128, K), jnp.float32),
            pltpu.VMEM((FS, 8, 8, 128), jnp.float32),
            pltpu.VMEM((FS, 8, 8, 128), jnp.float32),
            pltpu.SemaphoreType.DMA,
            pltpu.SemaphoreType.DMA,
            pltpu.SemaphoreType.DMA,
            pltpu.SemaphoreType.DMA,
        ],
    )
    out5 = f(idx, nf, cat_table, net)
    return out5.transpose(2, 4, 0, 1, 3).reshape(B, NR, K)


# speedup vs baseline: 1.3795x; 1.3522x over previous
import jax
import jax.numpy as jnp
from jax import lax
from jax.experimental import pallas as pl
from jax.experimental.pallas import tpu as pltpu
from jax.experimental.pallas import tpu_sc as plsc

B = 16384
NF = 26
NN = 13
NR = NF + NN
K = 64

NC, NS = 2, 16
NW = NC * NS
BPW = B // NW
NBB = BPW // 128
FS = 2
NCATC = NBB * (NF // FS)
NNUMC = NBB * NN
IDX_ROWS = NF * NBB
NF_PW = NN * BPW
ROW_RUN = 8 * 128
OUT_STRIDE = 128 * ROW_RUN


def _transpose_block(rows_v, stage4, frel):
    iota = lax.iota(jnp.int32, 16)
    i_frel = jnp.full((16,), frel, jnp.int32)
    i_k8 = [(kc * 16 + iota) // 8 for kc in range(K // 16)]
    i_k0 = iota % 8

    def body(i, carry):
        b0 = i * 4
        for u in range(4):
            i_b = jnp.full((16,), b0 + u, jnp.int32)
            for kc in range(K // 16):
                v = rows_v[frel * 128 + b0 + u, pl.ds(kc * 16, 16)]
                plsc.store_scatter(stage4, [i_frel, i_k8[kc], i_k0, i_b], v)
        return carry

    lax.fori_loop(0, 32, body, 0)


def _sc_body(idx_hbm, nf_hbm, table_hbm, net_hbm, out_hbm,
             idx_v, nf_v, net_v, rows0, rows1, stage0, stage1,
             gs0, gs1, os0, os1):
    wid = lax.axis_index("s") * NC + lax.axis_index("c")
    pltpu.sync_copy(net_hbm, net_v)
    pltpu.sync_copy(idx_hbm.at[pl.ds(wid * IDX_ROWS, IDX_ROWS), :], idx_v)
    pltpu.sync_copy(nf_hbm.at[pl.ds(wid * NF_PW, NF_PW)], nf_v)
    rows = (rows0, rows1)
    stages = (stage0, stage1)
    gsems = (gs0, gs1)
    osems = (os0, os1)

    def cat_gathers(t, slot):
        bbl = t // (NF // FS)
        fc = t % (NF // FS)
        return [
            pltpu.make_async_copy(
                table_hbm.at[idx_v.at[(fc * FS + frel) * NBB + bbl]],
                rows[slot].at[pl.ds(frel * 128, 128), :],
                gsems[slot],
            )
            for frel in range(FS)
        ]

    def cat_out(f0, bbg, stage4, osem):
        return pltpu.make_async_copy(
            stage4.at[:, :, :, pl.ds(0, 128)],
            out_hbm.at[pl.ds(f0, FS), :, bbg, :, :],
            osem,
        )

    for cp in cat_gathers(0, 0):
        cp.start()
    for cp in cat_gathers(1, 1):
        cp.start()

    def cat_chunk(i, slot):
        t = 2 * i + slot
        bbl = t // (NF // FS)
        fc = t % (NF // FS)
        f0 = fc * FS
        bbg = wid * NBB + bbl
        for cp in cat_gathers(t, slot):
            cp.wait()

        @pl.when(i >= 1)
        def _():
            pltpu.make_async_copy(
                out_hbm.at[pl.ds(0, FS), :, 0, :, :],
                stages[slot].at[:, :, :, pl.ds(0, 128)],
                osems[slot]).wait()

        for frel in range(FS):
            _transpose_block(rows[slot], stages[slot], frel)
        cat_out(f0, bbg, stages[slot], osems[slot]).start()

        @pl.when(t + 2 < NCATC)
        def _():
            for cp in cat_gathers(t + 2, slot):
                cp.start()

    def cat_loop(i, carry):
        cat_chunk(i, 0)
        cat_chunk(i, 1)
        return carry

    lax.fori_loop(0, NCATC // 2, cat_loop, 0)
    for slot in range(2):
        pltpu.make_async_copy(
            out_hbm.at[pl.ds(0, FS), :, 0, :, :],
            stages[slot].at[:, :, :, pl.ds(0, 128)],
            osems[slot]).wait()

    def num_chunk(i, slot):
        t = 2 * i + slot
        bbl = t // NN
        j = t % NN
        bbg = wid * NBB + bbl
        stage2 = stages[slot]

        @pl.when(i >= 1)
        def _():
            pltpu.make_async_copy(
                out_hbm.at[0, :, 0, :, :],
                stage2.at[0, :, :, pl.ds(0, 128)], osems[slot]).wait()

        netws = [net_v[pl.ds(j * K + kc * 16, 16)] for kc in range(K // 16)]

        def nbody(b16, carry):
            nfvec = nf_v[pl.ds(j * BPW + bbl * 128 + b16 * 16, 16)]
            for k in range(K):
                stage2[0, k // 8, k % 8, pl.ds(b16 * 16, 16)] = (
                    netws[k // 16][k % 16] * nfvec)
            return carry

        lax.fori_loop(0, 8, nbody, 0)
        pltpu.make_async_copy(
            stage2.at[0, :, :, pl.ds(0, 128)],
            out_hbm.at[NF + j, :, bbg, :, :],
            osems[slot]).start()

    def num_loop(i, carry):
        num_chunk(i, 0)
        num_chunk(i, 1)
        return carry

    lax.fori_loop(0, NNUMC // 2, num_loop, 0)
    for slot in range(2):
        pltpu.make_async_copy(
            out_hbm.at[0, :, 0, :, :],
            stages[slot].at[0, :, :, pl.ds(0, 128)], osems[slot]).wait()


def kernel(cat_features, num_features, cat_table, num_embedding):
    cat = cat_features.astype(jnp.int32)
    idx = (cat.T.reshape(NF, NW, NBB, 128)
           .transpose(1, 0, 2, 3).reshape(NW * IDX_ROWS, 128))
    nf = (num_features.T.reshape(NN, NW, NBB * 128)
          .transpose(1, 0, 2).reshape(NW * NF_PW))
    net = num_embedding.T.reshape(NN * K)
    mesh = plsc.VectorSubcoreMesh(core_axis_name="c", subcore_axis_name="s")
    f = pl.kernel(
        _sc_body,
        out_type=jax.ShapeDtypeStruct((NR, 8, B // 128, 8, 128),
                                      jnp.float32),
        mesh=mesh,
        compiler_params=pltpu.CompilerParams(
            use_tc_tiling_on_sc=False, needs_layout_passes=False),
        scratch_types=[
            pltpu.VMEM((IDX_ROWS, 128), jnp.int32),
            pltpu.VMEM((NF_PW,), jnp.float32),
            pltpu.VMEM((NN * K,), jnp.float32),
            pltpu.VMEM((FS * 128, K), jnp.float32),
            pltpu.VMEM((FS * 128, K), jnp.float32),
            pltpu.VMEM((FS, 8, 8, 130), jnp.float32),
            pltpu.VMEM((FS, 8, 8, 130), jnp.float32),
            pltpu.SemaphoreType.DMA,
            pltpu.SemaphoreType.DMA,
            pltpu.SemaphoreType.DMA,
            pltpu.SemaphoreType.DMA,
        ],
    )
    out5 = f(idx, nf, cat_table, net)
    return out5.transpose(2, 4, 0, 1, 3).reshape(B, NR, K)
